# baseline (speedup 1.0000x reference)
import functools

import jax
import jax.numpy as jnp
from jax import lax
from jax.experimental import pallas as pl
from jax.experimental.pallas import tpu as pltpu
from jax.experimental.pallas import tpu_sc as plsc

B, T, M, D, NH = 4, 1024, 1024, 768, 12
S = T + M
DH = D // NH
F32 = jnp.float32
I32 = jnp.int32

_pallas_call = pl.pallas_call

_PC = 512


def _prep_body(age_ref, aget_ref, mod_ref, fmi_ref, gsrc_ref):
    b = pl.program_id(0)
    rsel = (lax.broadcasted_iota(I32, (B, 1), 0) == b).astype(F32)
    aa = jnp.sum(age_ref[...] * rsel, axis=0, keepdims=True)
    am = jnp.sum(mod_ref[...] * rsel, axis=0, keepdims=True)
    a0 = jnp.concatenate([am, aa], axis=1)
    csel = (lax.broadcasted_iota(I32, (1, B), 1) == b).astype(F32)
    ak = jnp.sum(aget_ref[...] * csel, axis=1, keepdims=True)
    kidx = lax.broadcasted_iota(I32, (T, 1), 0)

    d1 = jnp.zeros((T, 1), F32)
    for c in range(S // _PC):
        a0c = a0[:, c * _PC:(c + 1) * _PC]
        idx = c * _PC + lax.broadcasted_iota(I32, (1, _PC), 1)
        lt = (a0c < ak).astype(F32)
        eq = ((a0c == ak) & (idx < M + kidx)).astype(F32)
        d1 = d1 + jnp.sum(lt + eq, axis=1, keepdims=True)

    parts = []
    for c in range(S // _PC):
        sidx = (c * _PC + lax.broadcasted_iota(I32, (1, _PC), 1)).astype(F32)
        le = (d1 <= sidx).astype(F32)
        parts.append(jnp.sum(le, axis=0, keepdims=True))
    cum = jnp.concatenate(parts, axis=1)
    cumprev = jnp.concatenate([jnp.zeros((1, 1), F32), cum[:, :S - 1]], axis=1)
    fmi = (cum - cumprev).astype(I32)
    s_full = lax.broadcasted_iota(I32, (1, S), 1).astype(F32)
    src = jnp.where(fmi == 1, (M - 1) + cum, s_full - cum)
    fmi_ref[0] = fmi
    gsrc_ref[0] = b * S + src.astype(I32)


def _prep_call(age, age_t, mod_age):
    full2 = lambda a: pl.BlockSpec(a.shape, lambda b: (0, 0))
    fmi, gsrc = _pallas_call(
        _prep_body,
        grid=(B,),
        in_specs=[full2(age), full2(age_t), full2(mod_age)],
        out_specs=[pl.BlockSpec((1, 1, S), lambda b: (b, 0, 0)),
                   pl.BlockSpec((1, 1, S), lambda b: (b, 0, 0))],
        out_shape=[jax.ShapeDtypeStruct((B, 1, S), I32),
                   jax.ShapeDtypeStruct((B, 1, S), I32)],
    )(age, age_t, mod_age)
    return fmi.reshape(B, S), gsrc.reshape(B, S)


_BM = 512


def _layernorm(xb, g, bta):
    mu = jnp.mean(xb, axis=1, keepdims=True)
    var = jnp.mean((xb - mu) * (xb - mu), axis=1, keepdims=True)
    return (xb - mu) * lax.rsqrt(var + 1e-5) * g + bta


def _qkv_body(x_ref, g_ref, bt_ref, w_ref, bias_ref, o_ref):
    h = _layernorm(x_ref[...], g_ref[...], bt_ref[...])
    o_ref[...] = jnp.dot(h, w_ref[...], preferred_element_type=F32) + bias_ref[...]


def _qkv_call(u2d, g, bta, w, bias):
    mb, nb = (B * S) // _BM, 3
    bn = (3 * D) // nb
    return _pallas_call(
        _qkv_body,
        grid=(mb, nb),
        in_specs=[pl.BlockSpec((_BM, D), lambda i, j: (i, 0)),
                  pl.BlockSpec((1, D), lambda i, j: (0, 0)),
                  pl.BlockSpec((1, D), lambda i, j: (0, 0)),
                  pl.BlockSpec((D, bn), lambda i, j: (0, j)),
                  pl.BlockSpec((1, bn), lambda i, j: (0, j))],
        out_specs=pl.BlockSpec((_BM, bn), lambda i, j: (i, j)),
        out_shape=jax.ShapeDtypeStruct((B * S, 3 * D), F32),
    )(u2d, g, bta, w, bias)


_BQ = 512


def _attn_body(arow_ref, acol_ref, tcol_ref, q_ref, k_ref, v_ref, o_ref):
    qv = q_ref[0, 0]
    kv = k_ref[0, 0]
    s = lax.dot_general(qv, kv, (((1,), (1,)), ((), ())),
                        preferred_element_type=F32) * (1.0 / 8.0)
    ak = arow_ref[0]
    aq = acol_ref[0]
    tq = tcol_ref[0]
    allowed = (ak <= tq) & (ak != -10000.0) & (aq != -10000.0)
    s = jnp.where(allowed, s, -1e9)
    m = jnp.max(s, axis=1, keepdims=True)
    p = jnp.exp(s - m)
    l = jnp.sum(p, axis=1, keepdims=True)
    o_ref[0, 0] = jnp.dot(p, v_ref[0, 0], preferred_element_type=F32) / l


def _attn_call(qkv4d, a_u, a_col, t_col):
    return _pallas_call(
        _attn_body,
        grid=(B, NH, S // _BQ),
        in_specs=[pl.BlockSpec((1, 1, S), lambda b, h, i: (b, 0, 0)),
                  pl.BlockSpec((1, _BQ, 1), lambda b, h, i: (b, i, 0)),
                  pl.BlockSpec((1, _BQ, 1), lambda b, h, i: (b, i, 0)),
                  pl.BlockSpec((1, 1, _BQ, DH), lambda b, h, i: (b, h, i, 0)),
                  pl.BlockSpec((1, 1, S, DH), lambda b, h, i: (b, NH + h, 0, 0)),
                  pl.BlockSpec((1, 1, S, DH), lambda b, h, i: (b, 2 * NH + h, 0, 0))],
        out_specs=pl.BlockSpec((1, 1, _BQ, DH), lambda b, h, i: (b, h, i, 0)),
        out_shape=jax.ShapeDtypeStruct((B, NH, S, DH), F32),
    )(a_u, a_col, t_col, qkv4d, qkv4d, qkv4d)


def _proj_body(y_ref, w_ref, bias_ref, res_ref, o_ref):
    o_ref[...] = (res_ref[...] + bias_ref[...]
                  + jnp.dot(y_ref[...], w_ref[...], preferred_element_type=F32))


def _proj_call(y2d, w, bias, res):
    return _pallas_call(
        _proj_body,
        grid=((B * S) // _BM,),
        in_specs=[pl.BlockSpec((_BM, D), lambda i: (i, 0)),
                  pl.BlockSpec((D, D), lambda i: (0, 0)),
                  pl.BlockSpec((1, D), lambda i: (0, 0)),
                  pl.BlockSpec((_BM, D), lambda i: (i, 0))],
        out_specs=pl.BlockSpec((_BM, D), lambda i: (i, 0)),
        out_shape=jax.ShapeDtypeStruct((B * S, D), F32),
    )(y2d, w, bias, res)


def _mlp_body(x_ref, g_ref, bt_ref, wfc_ref, bfc_ref, wout_ref, bout_ref, o_ref):
    xb = x_ref[...]
    h2 = _layernorm(xb, g_ref[...], bt_ref[...])
    g = jnp.dot(h2, wfc_ref[...], preferred_element_type=F32) + bfc_ref[...]
    g = 0.5 * g * (1.0 + jnp.tanh(0.7978845608028654 * (g + 0.044715 * g * g * g)))
    o_ref[...] = (xb + bout_ref[...]
                  + jnp.dot(g, wout_ref[...], preferred_element_type=F32))


def _mlp_call(x2d, g, bta, w_fc, b_fc, w_out, b_out):
    return _pallas_call(
        _mlp_body,
        grid=((B * S) // _BM,),
        in_specs=[pl.BlockSpec((_BM, D), lambda i: (i, 0)),
                  pl.BlockSpec((1, D), lambda i: (0, 0)),
                  pl.BlockSpec((1, D), lambda i: (0, 0)),
                  pl.BlockSpec((D, 4 * D), lambda i: (0, 0)),
                  pl.BlockSpec((1, 4 * D), lambda i: (0, 0)),
                  pl.BlockSpec((4 * D, D), lambda i: (0, 0)),
                  pl.BlockSpec((1, D), lambda i: (0, 0))],
        out_specs=pl.BlockSpec((_BM, D), lambda i: (i, 0)),
        out_shape=jax.ShapeDtypeStruct((B * S, D), F32),
    )(x2d, g, bta, w_fc, b_fc, w_out, b_out)


_NC, _NS = 2, 16
_NW = _NC * _NS
_ROWS = (B * S) // _NW
_CH = 128


@functools.lru_cache(maxsize=1)
def _build_sc_permute():
    @functools.partial(
        pl.kernel,
        mesh=plsc.VectorSubcoreMesh(core_axis_name="c", subcore_axis_name="s"),
        out_type=jax.ShapeDtypeStruct((B * S, D), F32),
        scratch_types=[pltpu.VMEM((_CH,), I32),
                       pltpu.VMEM((_CH, D), F32),
                       pltpu.SemaphoreType.DMA],
    )
    def _sc_permute_kernel(tab_hbm, idx_hbm, out_hbm, idx_v, rows_v, sem):
        wid = lax.axis_index("s") * _NC + lax.axis_index("c")
        for c in range(_ROWS // _CH):
            base = wid * _ROWS + c * _CH
            pltpu.sync_copy(idx_hbm.at[pl.ds(base, _CH)], idx_v)
            pltpu.async_copy(tab_hbm.at[idx_v], rows_v, sem).wait()
            pltpu.sync_copy(rows_v, out_hbm.at[pl.ds(base, _CH)])

    return _sc_permute_kernel


def _sc_permute(x2d, gidx):
    return _build_sc_permute()(x2d, gidx)


def kernel(x, age, targets_age, mod_idx, mod_age, mod_emb_0,
           ln1_g, ln1_b, w_qkv, b_qkv, w_proj, b_proj,
           ln2_g, ln2_b, w_fc, b_fc, w_out, b_out):
    u2d = jnp.concatenate([mod_emb_0.reshape(B, M, D), x], axis=1).reshape(B * S, D)
    a_u = jnp.concatenate([mod_age, age], axis=1)
    t_u = jnp.concatenate([mod_age, targets_age], axis=1)

    fmi, gsrc = _prep_call(age, jnp.transpose(age), mod_age)

    qkv = _qkv_call(u2d, ln1_g.reshape(1, D), ln1_b.reshape(1, D),
                    w_qkv, b_qkv.reshape(1, 3 * D))
    qkv4 = qkv.reshape(B, S, 3 * NH, DH).transpose(0, 2, 1, 3)
    y = _attn_call(qkv4, a_u.reshape(B, 1, S), a_u[..., None], t_u[..., None])
    y2d = y.transpose(0, 2, 1, 3).reshape(B * S, D)
    xo1 = _proj_call(y2d, w_proj, b_proj.reshape(1, D), u2d)
    xo2 = _mlp_call(xo1, ln2_g.reshape(1, D), ln2_b.reshape(1, D),
                    w_fc, b_fc.reshape(1, 4 * D), w_out, b_out.reshape(1, D))

    out = _sc_permute(xo2, gsrc.reshape(B * S))
    return out.reshape(B, S, D), fmi


# speedup vs baseline: 1.6776x; 1.6776x over previous
import functools

import jax
import jax.numpy as jnp
from jax import lax
from jax.experimental import pallas as pl
from jax.experimental.pallas import tpu as pltpu
from jax.experimental.pallas import tpu_sc as plsc

B, T, M, D, NH = 4, 1024, 1024, 768, 12
S = T + M
DH = D // NH
F32 = jnp.float32
I32 = jnp.int32

_pallas_call = pl.pallas_call

_PC = 512


def _prep_body(age_ref, aget_ref, mod_ref, fmi_ref, gsrc_ref):
    b = pl.program_id(0)
    rsel = (lax.broadcasted_iota(I32, (B, 1), 0) == b).astype(F32)
    aa = jnp.sum(age_ref[...] * rsel, axis=0, keepdims=True)
    am = jnp.sum(mod_ref[...] * rsel, axis=0, keepdims=True)
    a0 = jnp.concatenate([am, aa], axis=1)
    csel = (lax.broadcasted_iota(I32, (1, B), 1) == b).astype(F32)
    ak = jnp.sum(aget_ref[...] * csel, axis=1, keepdims=True)
    kidx = lax.broadcasted_iota(I32, (T, 1), 0)

    d1 = jnp.zeros((T, 1), F32)
    for c in range(S // _PC):
        a0c = a0[:, c * _PC:(c + 1) * _PC]
        idx = c * _PC + lax.broadcasted_iota(I32, (1, _PC), 1)
        lt = (a0c < ak).astype(F32)
        eq = ((a0c == ak) & (idx < M + kidx)).astype(F32)
        d1 = d1 + jnp.sum(lt + eq, axis=1, keepdims=True)

    parts = []
    for c in range(S // _PC):
        sidx = (c * _PC + lax.broadcasted_iota(I32, (1, _PC), 1)).astype(F32)
        le = (d1 <= sidx).astype(F32)
        parts.append(jnp.sum(le, axis=0, keepdims=True))
    cum = jnp.concatenate(parts, axis=1)
    cumprev = jnp.concatenate([jnp.zeros((1, 1), F32), cum[:, :S - 1]], axis=1)
    fmi = (cum - cumprev).astype(I32)
    s_full = lax.broadcasted_iota(I32, (1, S), 1).astype(F32)
    src = jnp.where(fmi == 1, (M - 1) + cum, s_full - cum)
    fmi_ref[0] = fmi
    gsrc_ref[0] = b * S + src.astype(I32)


def _prep_call(age, age_t, mod_age):
    full2 = lambda a: pl.BlockSpec(a.shape, lambda b: (0, 0))
    fmi, gsrc = _pallas_call(
        _prep_body,
        grid=(B,),
        in_specs=[full2(age), full2(age_t), full2(mod_age)],
        out_specs=[pl.BlockSpec((1, 1, S), lambda b: (b, 0, 0)),
                   pl.BlockSpec((1, 1, S), lambda b: (b, 0, 0))],
        out_shape=[jax.ShapeDtypeStruct((B, 1, S), I32),
                   jax.ShapeDtypeStruct((B, 1, S), I32)],
    )(age, age_t, mod_age)
    return fmi.reshape(B, S), gsrc.reshape(B, S)


_BM = 512


def _layernorm(xb, g, bta):
    mu = jnp.mean(xb, axis=1, keepdims=True)
    var = jnp.mean((xb - mu) * (xb - mu), axis=1, keepdims=True)
    return (xb - mu) * lax.rsqrt(var + 1e-5) * g + bta


def _qkv_body(x_ref, g_ref, bt_ref, w_ref, bias_ref, o_ref):
    h = _layernorm(x_ref[...], g_ref[...], bt_ref[...])
    o_ref[...] = jnp.dot(h, w_ref[...], preferred_element_type=F32) + bias_ref[...]


def _qkv_call(u2d, g, bta, w, bias):
    mb, nb = (B * S) // _BM, 3
    bn = (3 * D) // nb
    return _pallas_call(
        _qkv_body,
        grid=(mb, nb),
        in_specs=[pl.BlockSpec((_BM, D), lambda i, j: (i, 0)),
                  pl.BlockSpec((1, D), lambda i, j: (0, 0)),
                  pl.BlockSpec((1, D), lambda i, j: (0, 0)),
                  pl.BlockSpec((D, bn), lambda i, j: (0, j)),
                  pl.BlockSpec((1, bn), lambda i, j: (0, j))],
        out_specs=pl.BlockSpec((_BM, bn), lambda i, j: (i, j)),
        out_shape=jax.ShapeDtypeStruct((B * S, 3 * D), F32),
    )(u2d, g, bta, w, bias)


_BQ = 512


_HP = NH // 2


def _attn_body(arow_ref, acol_ref, tcol_ref, q_ref, k_ref, v_ref, o_ref):
    q2 = q_ref[0]
    k2 = k_ref[0]
    v2 = v_ref[0]
    ak = arow_ref[0]
    aq = acol_ref[0]
    tq = tcol_ref[0]
    akx = jnp.where(ak != -10000.0, ak, 3.0e38)
    tqx = jnp.where(aq != -10000.0, tq, -3.0e38)
    allowed = akx <= tqx
    lq = lax.broadcasted_iota(I32, (_BQ, 2 * DH), 1) < DH
    lv = lax.broadcasted_iota(I32, (S, 2 * DH), 1) < DH
    q0 = jnp.where(lq, q2, 0.0)
    q1 = jnp.where(lq, 0.0, q2)
    v0 = jnp.where(lv, v2, 0.0)
    v1 = jnp.where(lv, 0.0, v2)

    def one_head(qh, vh):
        s = lax.dot_general(qh, k2, (((1,), (1,)), ((), ())),
                            preferred_element_type=F32) * 0.125
        s = jnp.where(allowed, s, -1e9)
        m = jnp.max(s, axis=1, keepdims=True)
        p = jnp.exp(s - m)
        r = 1.0 / jnp.sum(p, axis=1, keepdims=True)
        return jnp.dot(p, vh, preferred_element_type=F32) * r

    o_ref[0, 0] = one_head(q0, v0) + one_head(q1, v1)


def _attn_call(qkv3d, a_u, a_col, t_col):
    return _pallas_call(
        _attn_body,
        grid=(B, _HP, S // _BQ),
        in_specs=[pl.BlockSpec((1, 1, S), lambda b, hp, i: (b, 0, 0)),
                  pl.BlockSpec((1, _BQ, 1), lambda b, hp, i: (b, i, 0)),
                  pl.BlockSpec((1, _BQ, 1), lambda b, hp, i: (b, i, 0)),
                  pl.BlockSpec((1, _BQ, 2 * DH), lambda b, hp, i: (b, i, hp)),
                  pl.BlockSpec((1, S, 2 * DH), lambda b, hp, i: (b, 0, _HP + hp)),
                  pl.BlockSpec((1, S, 2 * DH), lambda b, hp, i: (b, 0, 2 * _HP + hp))],
        out_specs=pl.BlockSpec((1, 1, _BQ, 2 * DH), lambda b, hp, i: (b, hp, i, 0)),
        out_shape=jax.ShapeDtypeStruct((B, _HP, S, 2 * DH), F32),
    )(a_u, a_col, t_col, qkv3d, qkv3d, qkv3d)


_SB = S // _BM


def _proj_body(y_ref, w_ref, bias_ref, res_ref, o_ref):
    y = jnp.concatenate([y_ref[0, j] for j in range(_HP)], axis=1)
    o_ref[...] = (res_ref[...] + bias_ref[...]
                  + jnp.dot(y, w_ref[...], preferred_element_type=F32))


def _proj_call(y4d, w, bias, res):
    return _pallas_call(
        _proj_body,
        grid=((B * S) // _BM,),
        in_specs=[pl.BlockSpec((1, _HP, _BM, 2 * DH),
                               lambda i: (i // _SB, 0, i % _SB, 0)),
                  pl.BlockSpec((D, D), lambda i: (0, 0)),
                  pl.BlockSpec((1, D), lambda i: (0, 0)),
                  pl.BlockSpec((_BM, D), lambda i: (i, 0))],
        out_specs=pl.BlockSpec((_BM, D), lambda i: (i, 0)),
        out_shape=jax.ShapeDtypeStruct((B * S, D), F32),
    )(y4d, w, bias, res)


def _mlp_body(x_ref, g_ref, bt_ref, wfc_ref, bfc_ref, wout_ref, bout_ref, o_ref):
    xb = x_ref[...]
    h2 = _layernorm(xb, g_ref[...], bt_ref[...])
    g = jnp.dot(h2, wfc_ref[...], preferred_element_type=F32) + bfc_ref[...]
    g = 0.5 * g * (1.0 + jnp.tanh(0.7978845608028654 * (g + 0.044715 * g * g * g)))
    o_ref[...] = (xb + bout_ref[...]
                  + jnp.dot(g, wout_ref[...], preferred_element_type=F32))


def _mlp_call(x2d, g, bta, w_fc, b_fc, w_out, b_out):
    return _pallas_call(
        _mlp_body,
        grid=((B * S) // _BM,),
        in_specs=[pl.BlockSpec((_BM, D), lambda i: (i, 0)),
                  pl.BlockSpec((1, D), lambda i: (0, 0)),
                  pl.BlockSpec((1, D), lambda i: (0, 0)),
                  pl.BlockSpec((D, 4 * D), lambda i: (0, 0)),
                  pl.BlockSpec((1, 4 * D), lambda i: (0, 0)),
                  pl.BlockSpec((4 * D, D), lambda i: (0, 0)),
                  pl.BlockSpec((1, D), lambda i: (0, 0))],
        out_specs=pl.BlockSpec((_BM, D), lambda i: (i, 0)),
        out_shape=jax.ShapeDtypeStruct((B * S, D), F32),
    )(x2d, g, bta, w_fc, b_fc, w_out, b_out)


_NC, _NS = 2, 16
_NW = _NC * _NS
_ROWS = (B * S) // _NW
_CH = 128


@functools.lru_cache(maxsize=1)
def _build_sc_permute():
    @functools.partial(
        pl.kernel,
        mesh=plsc.VectorSubcoreMesh(core_axis_name="c", subcore_axis_name="s"),
        out_type=jax.ShapeDtypeStruct((B * S, D), F32),
        scratch_types=[pltpu.VMEM((_CH,), I32),
                       pltpu.VMEM((_CH, D), F32),
                       pltpu.SemaphoreType.DMA],
    )
    def _sc_permute_kernel(tab_hbm, idx_hbm, out_hbm, idx_v, rows_v, sem):
        wid = lax.axis_index("s") * _NC + lax.axis_index("c")
        for c in range(_ROWS // _CH):
            base = wid * _ROWS + c * _CH
            pltpu.sync_copy(idx_hbm.at[pl.ds(base, _CH)], idx_v)
            pltpu.async_copy(tab_hbm.at[idx_v], rows_v, sem).wait()
            pltpu.sync_copy(rows_v, out_hbm.at[pl.ds(base, _CH)])

    return _sc_permute_kernel


def _sc_permute(x2d, gidx):
    return _build_sc_permute()(x2d, gidx)


def kernel(x, age, targets_age, mod_idx, mod_age, mod_emb_0,
           ln1_g, ln1_b, w_qkv, b_qkv, w_proj, b_proj,
           ln2_g, ln2_b, w_fc, b_fc, w_out, b_out):
    u2d = jnp.concatenate([mod_emb_0.reshape(B, M, D), x], axis=1).reshape(B * S, D)
    a_u = jnp.concatenate([mod_age, age], axis=1)
    t_u = jnp.concatenate([mod_age, targets_age], axis=1)

    fmi, gsrc = _prep_call(age, jnp.transpose(age), mod_age)

    qkv = _qkv_call(u2d, ln1_g.reshape(1, D), ln1_b.reshape(1, D),
                    w_qkv, b_qkv.reshape(1, 3 * D))
    y4 = _attn_call(qkv.reshape(B, S, 3 * D), a_u.reshape(B, 1, S),
                    a_u[..., None], t_u[..., None])
    xo1 = _proj_call(y4, w_proj, b_proj.reshape(1, D), u2d)
    xo2 = _mlp_call(xo1, ln2_g.reshape(1, D), ln2_b.reshape(1, D),
                    w_fc, b_fc.reshape(1, 4 * D), w_out, b_out.reshape(1, D))

    out = _sc_permute(xo2, gsrc.reshape(B * S))
    return out.reshape(B, S, D), fmi
